# baseline (speedup 1.0000x reference)
import functools

import jax
import jax.numpy as jnp
from jax import lax
from jax.experimental import pallas as pl
from jax.experimental.pallas import tpu as pltpu
from jax.experimental.pallas import tpu_sc as plsc

B = 16384
D = 128
NC = 2
NS = 16
NW = NC * NS

CHUNKS = 4
CB = B // CHUNKS
BPW = CB // NW

H1, H2, H3 = 512, 256, 128
TB = 2048


def _make_gather_body(chunk_base):
    def _gather_body(uid_hbm, iid_hbm, utab_hbm, itab_hbm, u_out, i_out,
                     uidx_v, iidx_v, urows_v, irows_v, usem, isem):
        wid = lax.axis_index("s") * NC + lax.axis_index("c")
        in_sl = pl.ds(chunk_base + wid * BPW, BPW)
        out_sl = pl.ds(wid * BPW, BPW)
        pltpu.sync_copy(uid_hbm.at[in_sl], uidx_v)
        ucp = pltpu.async_copy(utab_hbm.at[uidx_v], urows_v, usem)
        pltpu.sync_copy(iid_hbm.at[in_sl], iidx_v)
        icp = pltpu.async_copy(itab_hbm.at[iidx_v], irows_v, isem)
        ucp.wait()
        pltpu.sync_copy(urows_v, u_out.at[out_sl])
        icp.wait()
        pltpu.sync_copy(irows_v, i_out.at[out_sl])
    return _gather_body


@functools.cache
def _gather2(chunk_base):
    return pl.kernel(
        _make_gather_body(chunk_base),
        out_type=[
            jax.ShapeDtypeStruct((CB, D), jnp.float32),
            jax.ShapeDtypeStruct((CB, D), jnp.float32),
        ],
        mesh=plsc.VectorSubcoreMesh(core_axis_name="c", subcore_axis_name="s"),
        scratch_types=[
            pltpu.VMEM((BPW,), jnp.int32),
            pltpu.VMEM((BPW,), jnp.int32),
            pltpu.VMEM((BPW, D), jnp.float32),
            pltpu.VMEM((BPW, D), jnp.float32),
            pltpu.SemaphoreType.DMA,
            pltpu.SemaphoreType.DMA,
        ],
    )


def _mlp_body(u_ref, i_ref, w1_ref, b1_ref, w2_ref, b2_ref,
              w3_ref, b3_ref, w4t_ref, b4_ref, out_ref):
    f32, bf16 = jnp.float32, jnp.bfloat16
    zero = jnp.bfloat16(0.0)
    x = jnp.concatenate([u_ref[...], i_ref[...]], axis=1).astype(bf16)
    h = jnp.dot(x, w1_ref[...], preferred_element_type=f32).astype(bf16)
    h = jnp.maximum(h + b1_ref[...], zero)
    h = jnp.maximum(
        jnp.dot(h, w2_ref[...], preferred_element_type=f32).astype(bf16)
        + b2_ref[...], zero)
    h = jnp.maximum(
        jnp.dot(h, w3_ref[...], preferred_element_type=f32).astype(bf16)
        + b3_ref[...], zero)
    ot = lax.dot_general(w4t_ref[...], h,
                         (((1,), (1,)), ((), ())),
                         preferred_element_type=f32)
    out_ref[...] = (ot + b4_ref[...])[0]


def _mlp_body_acc(dst_ref, u_ref, i_ref, w1_ref, b1_ref, w2_ref, b2_ref,
                  w3_ref, b3_ref, w4t_ref, b4_ref, out_ref):
    del dst_ref
    _mlp_body(u_ref, i_ref, w1_ref, b1_ref, w2_ref, b2_ref,
              w3_ref, b3_ref, w4t_ref, b4_ref, out_ref)


def _mlp(dst, u_emb, i_emb, w1, b1, w2, b2, w3, b3, w4t, b4, block_off,
         interpret=False):
    const = lambda shape: pl.BlockSpec(shape, lambda n: (0, 0))
    out_spec = pl.BlockSpec((TB,), lambda n: (n + block_off,))
    return pl.pallas_call(
        _mlp_body_acc,
        grid=(CB // TB,),
        in_specs=[
            out_spec,
            pl.BlockSpec((TB, D), lambda n: (n, 0)),
            pl.BlockSpec((TB, D), lambda n: (n, 0)),
            const((2 * D, H1)),
            const((1, H1)),
            const((H1, H2)),
            const((1, H2)),
            const((H2, H3)),
            const((1, H3)),
            const((1, H3)),
            const((1, 1)),
        ],
        out_specs=out_spec,
        out_shape=jax.ShapeDtypeStruct((B,), jnp.float32),
        input_output_aliases={0: 0},
        interpret=interpret,
    )(dst, u_emb, i_emb, w1, b1, w2, b2, w3, b3, w4t, b4)


def kernel(user_ids, item_ids, user_table, item_table,
           W1, b1, W2, b2, W3, b3, W4, b4):
    bf16 = jnp.bfloat16
    uids = user_ids.astype(jnp.int32)
    iids = item_ids.astype(jnp.int32)
    wargs = (W1.astype(bf16), b1.reshape(1, -1).astype(bf16),
             W2.astype(bf16), b2.reshape(1, -1).astype(bf16),
             W3.astype(bf16), b3.reshape(1, -1).astype(bf16),
             W4.reshape(1, -1).astype(bf16), b4.reshape(1, 1))
    out = jnp.zeros((B,), jnp.float32)
    for c in range(CHUNKS):
        u_emb, i_emb = _gather2(c * CB)(uids, iids, user_table, item_table)
        out = _mlp(out, u_emb, i_emb, *wargs, block_off=c * (CB // TB))
    return out


# speedup vs baseline: 1.0874x; 1.0874x over previous
import functools

import jax
import jax.numpy as jnp
from jax import lax
from jax.experimental import pallas as pl
from jax.experimental.pallas import tpu as pltpu
from jax.experimental.pallas import tpu_sc as plsc

B = 16384
D = 128
NC = 2
NS = 16
NW = NC * NS

CHUNKS = 2
CB = B // CHUNKS
BPW = CB // NW

H1, H2, H3 = 512, 256, 128
TB = 4096


def _gather_body(uid_hbm, iid_hbm, utab_hbm, itab_hbm, u_out, i_out,
                 uidx_v, iidx_v, urows_v, irows_v, usem, isem):
    wid = lax.axis_index("s") * NC + lax.axis_index("c")
    sl = pl.ds(wid * BPW, BPW)
    pltpu.sync_copy(uid_hbm.at[sl], uidx_v)
    ucp = pltpu.async_copy(utab_hbm.at[uidx_v], urows_v, usem)
    pltpu.sync_copy(iid_hbm.at[sl], iidx_v)
    icp = pltpu.async_copy(itab_hbm.at[iidx_v], irows_v, isem)
    ucp.wait()
    pltpu.sync_copy(urows_v, u_out.at[sl])
    icp.wait()
    pltpu.sync_copy(irows_v, i_out.at[sl])


@functools.cache
def _gather2():
    return pl.kernel(
        _gather_body,
        out_type=[
            jax.ShapeDtypeStruct((CB, D), jnp.float32),
            jax.ShapeDtypeStruct((CB, D), jnp.float32),
        ],
        mesh=plsc.VectorSubcoreMesh(core_axis_name="c", subcore_axis_name="s"),
        scratch_types=[
            pltpu.VMEM((BPW,), jnp.int32),
            pltpu.VMEM((BPW,), jnp.int32),
            pltpu.VMEM((BPW, D), jnp.float32),
            pltpu.VMEM((BPW, D), jnp.float32),
            pltpu.SemaphoreType.DMA,
            pltpu.SemaphoreType.DMA,
        ],
    )


def _mlp_body(u_ref, i_ref, w1_ref, b1_ref, w2_ref, b2_ref,
              w3_ref, b3_ref, w4t_ref, b4_ref, out_ref):
    f32, bf16 = jnp.float32, jnp.bfloat16
    zero = jnp.bfloat16(0.0)
    x = jnp.concatenate([u_ref[...], i_ref[...]], axis=1).astype(bf16)
    h = jnp.dot(x, w1_ref[...], preferred_element_type=f32).astype(bf16)
    h = jnp.maximum(h + b1_ref[...], zero)
    h = jnp.maximum(
        jnp.dot(h, w2_ref[...], preferred_element_type=f32).astype(bf16)
        + b2_ref[...], zero)
    h = jnp.maximum(
        jnp.dot(h, w3_ref[...], preferred_element_type=f32).astype(bf16)
        + b3_ref[...], zero)
    ot = lax.dot_general(w4t_ref[...], h,
                         (((1,), (1,)), ((), ())),
                         preferred_element_type=f32)
    out_ref[...] = (ot + b4_ref[...])[0]


def _mlp_body_acc(dst_ref, u_ref, i_ref, w1_ref, b1_ref, w2_ref, b2_ref,
                  w3_ref, b3_ref, w4t_ref, b4_ref, out_ref):
    del dst_ref
    _mlp_body(u_ref, i_ref, w1_ref, b1_ref, w2_ref, b2_ref,
              w3_ref, b3_ref, w4t_ref, b4_ref, out_ref)


def _mlp(dst, u_emb, i_emb, w1, b1, w2, b2, w3, b3, w4t, b4, block_off,
         interpret=False):
    const = lambda shape: pl.BlockSpec(shape, lambda n: (0, 0))
    out_spec = pl.BlockSpec((TB,), lambda n: (n + block_off,))
    return pl.pallas_call(
        _mlp_body_acc,
        grid=(CB // TB,),
        in_specs=[
            out_spec,
            pl.BlockSpec((TB, D), lambda n: (n, 0)),
            pl.BlockSpec((TB, D), lambda n: (n, 0)),
            const((2 * D, H1)),
            const((1, H1)),
            const((H1, H2)),
            const((1, H2)),
            const((H2, H3)),
            const((1, H3)),
            const((1, H3)),
            const((1, 1)),
        ],
        out_specs=out_spec,
        out_shape=jax.ShapeDtypeStruct((B,), jnp.float32),
        input_output_aliases={0: 0},
        interpret=interpret,
    )(dst, u_emb, i_emb, w1, b1, w2, b2, w3, b3, w4t, b4)


def kernel(user_ids, item_ids, user_table, item_table,
           W1, b1, W2, b2, W3, b3, W4, b4):
    bf16 = jnp.bfloat16
    uids = user_ids.astype(jnp.int32)
    iids = item_ids.astype(jnp.int32)
    wargs = (W1.astype(bf16), b1.reshape(1, -1).astype(bf16),
             W2.astype(bf16), b2.reshape(1, -1).astype(bf16),
             W3.astype(bf16), b3.reshape(1, -1).astype(bf16),
             W4.reshape(1, -1).astype(bf16), b4.reshape(1, 1))
    out = jnp.zeros((B,), jnp.float32)
    for c in range(CHUNKS):
        sl = slice(c * CB, (c + 1) * CB)
        u_emb, i_emb = _gather2()(uids[sl], iids[sl], user_table, item_table)
        out = _mlp(out, u_emb, i_emb, *wargs, block_off=c * (CB // TB))
    return out


# speedup vs baseline: 1.1176x; 1.0278x over previous
import functools

import jax
import jax.numpy as jnp
from jax import lax
from jax.experimental import pallas as pl
from jax.experimental.pallas import tpu as pltpu
from jax.experimental.pallas import tpu_sc as plsc

B = 16384
D = 128
NC = 2
NS = 16
NW = NC * NS

CHUNKS = 2
CB = B // CHUNKS
BPW = CB // NW

H1, H2, H3 = 512, 256, 128
TB = 4096


def _make_gather_body(chunk_base):
    def _gather_body(uid_hbm, iid_hbm, utab_hbm, itab_hbm, u_out, i_out,
                     uidx_v, iidx_v, urows_v, irows_v, usem, isem):
        wid = lax.axis_index("s") * NC + lax.axis_index("c")
        in_sl = pl.ds(chunk_base + wid * BPW, BPW)
        out_sl = pl.ds(wid * BPW, BPW)
        pltpu.sync_copy(uid_hbm.at[in_sl], uidx_v)
        ucp = pltpu.async_copy(utab_hbm.at[uidx_v], urows_v, usem)
        pltpu.sync_copy(iid_hbm.at[in_sl], iidx_v)
        icp = pltpu.async_copy(itab_hbm.at[iidx_v], irows_v, isem)
        ucp.wait()
        pltpu.sync_copy(urows_v, u_out.at[out_sl])
        icp.wait()
        pltpu.sync_copy(irows_v, i_out.at[out_sl])
    return _gather_body


@functools.cache
def _gather2(chunk_base):
    return pl.kernel(
        _make_gather_body(chunk_base),
        out_type=[
            jax.ShapeDtypeStruct((CB, D), jnp.float32),
            jax.ShapeDtypeStruct((CB, D), jnp.float32),
        ],
        mesh=plsc.VectorSubcoreMesh(core_axis_name="c", subcore_axis_name="s"),
        scratch_types=[
            pltpu.VMEM((BPW,), jnp.int32),
            pltpu.VMEM((BPW,), jnp.int32),
            pltpu.VMEM((BPW, D), jnp.float32),
            pltpu.VMEM((BPW, D), jnp.float32),
            pltpu.SemaphoreType.DMA,
            pltpu.SemaphoreType.DMA,
        ],
    )


def _mlp_body(u_ref, i_ref, w1_ref, b1_ref, w2_ref, b2_ref,
              w3_ref, b3_ref, w4t_ref, b4_ref, out_ref):
    f32, bf16 = jnp.float32, jnp.bfloat16
    zero = jnp.bfloat16(0.0)
    x = jnp.concatenate([u_ref[...], i_ref[...]], axis=1).astype(bf16)
    h = jnp.dot(x, w1_ref[...], preferred_element_type=f32).astype(bf16)
    h = jnp.maximum(h + b1_ref[...], zero)
    h = jnp.maximum(
        jnp.dot(h, w2_ref[...], preferred_element_type=f32).astype(bf16)
        + b2_ref[...], zero)
    h = jnp.maximum(
        jnp.dot(h, w3_ref[...], preferred_element_type=f32).astype(bf16)
        + b3_ref[...], zero)
    ot = lax.dot_general(w4t_ref[...], h,
                         (((1,), (1,)), ((), ())),
                         preferred_element_type=f32)
    out_ref[...] = (ot + b4_ref[...])[0]


def _mlp_body_acc(dst_ref, u_ref, i_ref, w1_ref, b1_ref, w2_ref, b2_ref,
                  w3_ref, b3_ref, w4t_ref, b4_ref, out_ref):
    del dst_ref
    _mlp_body(u_ref, i_ref, w1_ref, b1_ref, w2_ref, b2_ref,
              w3_ref, b3_ref, w4t_ref, b4_ref, out_ref)


def _mlp(dst, u_emb, i_emb, w1, b1, w2, b2, w3, b3, w4t, b4, block_off,
         interpret=False):
    const = lambda shape: pl.BlockSpec(shape, lambda n: (0, 0))
    out_spec = pl.BlockSpec((TB,), lambda n: (n + block_off,))
    in_specs = [
        pl.BlockSpec((TB, D), lambda n: (n, 0)),
        pl.BlockSpec((TB, D), lambda n: (n, 0)),
        const((2 * D, H1)),
        const((1, H1)),
        const((H1, H2)),
        const((1, H2)),
        const((H2, H3)),
        const((1, H3)),
        const((1, H3)),
        const((1, 1)),
    ]
    args = (u_emb, i_emb, w1, b1, w2, b2, w3, b3, w4t, b4)
    if dst is not None:
        in_specs = [out_spec] + in_specs
        args = (dst,) + args
    return pl.pallas_call(
        _mlp_body_acc if dst is not None else _mlp_body,
        grid=(CB // TB,),
        in_specs=in_specs,
        out_specs=out_spec,
        out_shape=jax.ShapeDtypeStruct((B,), jnp.float32),
        input_output_aliases={0: 0} if dst is not None else {},
        interpret=interpret,
    )(*args)


def kernel(user_ids, item_ids, user_table, item_table,
           W1, b1, W2, b2, W3, b3, W4, b4):
    bf16 = jnp.bfloat16
    uids = user_ids.astype(jnp.int32)
    iids = item_ids.astype(jnp.int32)
    wargs = (W1.astype(bf16), b1.reshape(1, -1).astype(bf16),
             W2.astype(bf16), b2.reshape(1, -1).astype(bf16),
             W3.astype(bf16), b3.reshape(1, -1).astype(bf16),
             W4.reshape(1, -1).astype(bf16), b4.reshape(1, 1))
    out = None
    for c in range(CHUNKS):
        u_emb, i_emb = _gather2(c * CB)(uids, iids, user_table, item_table)
        out = _mlp(out, u_emb, i_emb, *wargs, block_off=c * (CB // TB))
    return out
